# baseline (speedup 1.0000x reference)
import functools

import jax
import jax.numpy as jnp
from jax import lax
from jax.experimental import pallas as pl
from jax.experimental.pallas import tpu as pltpu
from jax.experimental.pallas import tpu_sc as plsc

B, L, K, H, S, V, TG = 8, 2048, 512, 128, 8, 100000, 64
NT = B * L
NPAIR = B * S
_LANES = 16

_NC = 2
_NS = 16
_NW = _NC * _NS


_GCH = 128


def _make_sc_gather(nt):
    tok_per_w = nt // _NW
    ngch = tok_per_w // _GCH

    def body(embed_hbm, hist_hbm, post_hbm, auth_hbm, act_hbm,
             x0, x1, x2, x3, idx_v, rows_v, sem):
        wid = lax.axis_index("s") * _NC + lax.axis_index("c")
        base = wid * tok_per_w
        srcs = ((hist_hbm, x0), (post_hbm, x1), (auth_hbm, x2), (act_hbm, x3))
        for idx_hbm, out_hbm in srcs:
            for c in range(ngch):
                off = base + c * _GCH
                pltpu.sync_copy(idx_hbm.at[pl.ds(off, _GCH)], idx_v)
                pltpu.async_copy(embed_hbm.at[idx_v], rows_v, sem).wait()
                pltpu.sync_copy(rows_v, out_hbm.at[pl.ds(off, _GCH)])

    mesh = plsc.VectorSubcoreMesh(core_axis_name="c", subcore_axis_name="s")
    xt = jax.ShapeDtypeStruct((nt, H), jnp.float32)
    return functools.partial(
        pl.kernel, mesh=mesh,
        out_type=[xt, xt, xt, xt],
        compiler_params=pltpu.CompilerParams(needs_layout_passes=False),
        scratch_types=[
            pltpu.VMEM((_GCH,), jnp.int32),
            pltpu.VMEM((_GCH, H), jnp.float32),
            pltpu.SemaphoreType.DMA,
        ],
    )(body)


_BT = 2048


def _tc_mlp_body(x0, x1, x2, x3, gap, tgp, gamma, beta, w1, b1, w2, b2, out):
    oh = jnp.where(gap[...] == lax.broadcasted_iota(jnp.int32, (_BT, H), 1),
                   1.0, 0.0)
    x4 = jnp.dot(oh, tgp[...], preferred_element_type=jnp.float32,
                 precision=lax.Precision.HIGHEST)
    x = jnp.concatenate(
        [x0[...], x1[...], x2[...], x3[...], x4], axis=1)
    mu = jnp.mean(x, axis=-1, keepdims=True)
    var = jnp.mean((x - mu) ** 2, axis=-1, keepdims=True)
    xn = (x - mu) * lax.rsqrt(var + 1e-5) * gamma[...] + beta[...]
    h1 = jnp.dot(xn.astype(jnp.bfloat16), w1[...].astype(jnp.bfloat16),
                 preferred_element_type=jnp.float32) + b1[...]
    h1 = h1 * jax.nn.sigmoid(h1)
    ev = jnp.dot(h1.astype(jnp.bfloat16), w2[...].astype(jnp.bfloat16),
                 preferred_element_type=jnp.float32) + b2[...]
    out[...] = ev


def _tc_mlp(xs, gap2d, tgpad, ln_gamma, ln_beta, W1, b1, W2, b2, nt=NT):
    D5 = 5 * H
    D4 = 4 * H
    grid = (nt // _BT,)
    xspec = pl.BlockSpec((_BT, H), lambda i: (i, 0))

    def full(shape):
        return pl.BlockSpec(shape, lambda i: tuple(0 for _ in shape))

    return pl.pallas_call(
        _tc_mlp_body,
        grid=grid,
        in_specs=[xspec] * 4 + [pl.BlockSpec((_BT, 1), lambda i: (i, 0)),
                                full((H, H)), full((1, D5)), full((1, D5)),
                                full((D5, D4)), full((1, D4)),
                                full((D4, H)), full((1, H))],
        out_specs=pl.BlockSpec((_BT, H), lambda i: (i, 0)),
        out_shape=jax.ShapeDtypeStruct((nt, H), jnp.float32),
    )(*xs, gap2d, tgpad, ln_gamma.reshape(1, D5), ln_beta.reshape(1, D5),
      W1, b1.reshape(1, D4), W2, b2.reshape(1, H))


_PCH = 128
_NPCH = K // _PCH
_NVL = L // _LANES
_PAIRS_PER_W = NPAIR // _NW


def _sc_pack_body(gid_hbm, msk_hbm, event_hbm, pos_hbm, sid_hbm, emp_hbm,
                  states_hbm, mout_hbm,
                  ids_v, msk_v, idxl_v, chunk_v, pos_v, zbuf_v, sid_v,
                  emp_v, emp2_v, pos0_v, mko_v, sem):
    wid = lax.axis_index("s") * _NC + lax.axis_index("c")
    iota = lax.iota(jnp.int32, _LANES)
    zero16 = jnp.zeros((_LANES,), jnp.int32)
    fzero16 = jnp.zeros((_LANES,), jnp.float32)

    def zb_body(r, _):
        for j2 in range(H // _LANES):
            zbuf_v[r, pl.ds(j2 * _LANES, _LANES)] = fzero16
        return 0

    lax.fori_loop(0, _PCH, zb_body, 0)
    pltpu.sync_copy(pos_hbm.at[0], pos0_v)

    for p in range(_PAIRS_PER_W):
        pair = wid + p * _NW
        b = pair // S
        s = pair % S
        pltpu.sync_copy(gid_hbm.at[b], ids_v)
        pltpu.sync_copy(msk_hbm.at[b], msk_v)
        pltpu.sync_copy(sid_hbm.at[s], sid_v)
        pltpu.sync_copy(emp_hbm.at[s], emp_v)

        sval = s + 1

        def count_body(j, c):
            ids16 = ids_v[pl.ds(j * _LANES, _LANES)]
            mk16 = msk_v[pl.ds(j * _LANES, _LANES)]
            m = (ids16 == sval) & (mk16 != 0)
            return c + jnp.sum(jnp.where(m, 1, 0))

        count = lax.fori_loop(0, _NVL, count_body, jnp.int32(0))
        start = jnp.maximum(count - K, 0)
        n = count - start

        def zidx_body(j, _):
            idxl_v[pl.ds(j * _LANES, _LANES)] = j * _LANES + iota
            return 0

        lax.fori_loop(0, K // _LANES, zidx_body, 0)

        def rank_body(j, c):
            ids16 = ids_v[pl.ds(j * _LANES, _LANES)]
            mk16 = msk_v[pl.ds(j * _LANES, _LANES)]
            m = (ids16 == sval) & (mk16 != 0)
            mi = jnp.where(m, 1, 0)
            rank = plsc.cumsum(mi) + c - 1
            slot = rank - start
            wm = m & (slot >= 0)
            slot_c = jnp.maximum(slot, 0)
            gidx = b * L + j * _LANES + iota
            plsc.store_scatter(idxl_v, [slot_c], gidx, mask=wm)
            return c + jnp.sum(mi)

        lax.fori_loop(0, _NVL, rank_body, jnp.int32(0))

        for c4 in range(_NPCH):
            k0 = c4 * _PCH

            @pl.when(k0 < n)
            def _(c4=c4, k0=k0):
                pltpu.async_copy(event_hbm.at[idxl_v.at[pl.ds(k0, _PCH)]],
                                 chunk_v.at[c4], sem)

        for c4 in range(_NPCH):
            k0 = c4 * _PCH
            live = n - k0
            dst = states_hbm.at[pl.ds(pair * K + k0, _PCH)]

            def live_path(c4=c4, k0=k0, live=live, dst=dst):
                pltpu.make_async_copy(
                    event_hbm.at[idxl_v.at[pl.ds(k0, _PCH)]],
                    chunk_v.at[c4], sem).wait()
                pltpu.sync_copy(pos_hbm.at[pl.ds(k0, _PCH)], pos_v)

                def row_fn(r, _):
                    multf = jnp.where(r < live, 1.0, 0.0)
                    for j2 in range(H // _LANES):
                        sl = pl.ds(j2 * _LANES, _LANES)
                        chunk_v[c4, r, sl] = (chunk_v[c4, r, sl]
                                              + pos_v[r, sl]
                                              + sid_v[sl]) * multf
                    return 0

                lax.fori_loop(0, _PCH, row_fn, 0)
                pltpu.sync_copy(chunk_v.at[c4], dst)

            if c4 == 0:
                @pl.when(n > 0)
                def _(live_path=live_path):
                    live_path()

                @pl.when(n == 0)
                def _(dst=dst):
                    pltpu.sync_copy(zbuf_v, dst)
                    for j2 in range(H // _LANES):
                        sl = pl.ds(j2 * _LANES, _LANES)
                        emp2_v[0, sl] = emp_v[sl] + pos0_v[sl] + sid_v[sl]
                    pltpu.sync_copy(emp2_v,
                                    states_hbm.at[pl.ds(pair * K, 1)])
            else:
                @pl.when(live > 0)
                def _(live_path=live_path):
                    live_path()

                @pl.when(live <= 0)
                def _(dst=dst):
                    pltpu.sync_copy(zbuf_v, dst)

        def mk_body(j, _):
            k16 = j * _LANES + iota
            mv = (k16 < n) | ((k16 == 0) & (n == 0))
            mko_v[pl.ds(j * _LANES, _LANES)] = jnp.where(mv, 1, 0)
            return 0

        lax.fori_loop(0, K // _LANES, mk_body, 0)
        pltpu.sync_copy(mko_v, mout_hbm.at[pair])


def _sc_pack(gid, maskI, event, pos_table, sid_rows, empty_tokens):
    mesh = plsc.VectorSubcoreMesh(core_axis_name="c", subcore_axis_name="s")
    fn = functools.partial(
        pl.kernel, mesh=mesh,
        out_type=[jax.ShapeDtypeStruct((NPAIR * K, H), jnp.float32),
                  jax.ShapeDtypeStruct((NPAIR, K), jnp.int32)],
        compiler_params=pltpu.CompilerParams(needs_layout_passes=False),
        scratch_types=[
            pltpu.VMEM((L,), jnp.int32),
            pltpu.VMEM((L,), jnp.int32),
            pltpu.VMEM((K,), jnp.int32),
            pltpu.VMEM((_NPCH, _PCH, H), jnp.float32),
            pltpu.VMEM((_PCH, H), jnp.float32),
            pltpu.VMEM((_PCH, H), jnp.float32),
            pltpu.VMEM((H,), jnp.float32),
            pltpu.VMEM((H,), jnp.float32),
            pltpu.VMEM((1, H), jnp.float32),
            pltpu.VMEM((H,), jnp.float32),
            pltpu.VMEM((K,), jnp.int32),
            pltpu.SemaphoreType.DMA,
        ],
    )(_sc_pack_body)
    return fn(gid, maskI, event, pos_table, sid_rows, empty_tokens)


def kernel(history_tokens, history_post_tokens, history_author_tokens,
           history_action_tokens, history_time_gap, history_group_ids,
           history_mask, embed_table, time_gap_table, seq_id_table, pos_table,
           ln_gamma, ln_beta, W1, b1, W2, b2, empty_tokens):
    hist = history_tokens.reshape(NT).astype(jnp.int32)
    post = history_post_tokens.reshape(NT).astype(jnp.int32)
    auth = history_author_tokens.reshape(NT).astype(jnp.int32)
    act = history_action_tokens.reshape(NT).astype(jnp.int32)
    gap = history_time_gap.reshape(NT).astype(jnp.int32)
    gid = history_group_ids.astype(jnp.int32)
    maskI = history_mask.astype(jnp.int32)

    tgpad = jnp.zeros((H, H), jnp.float32).at[:TG + 1].set(time_gap_table)
    xs = _make_sc_gather(NT)(embed_table, hist, post, auth, act)
    event = _tc_mlp(xs, gap.reshape(NT, 1), tgpad,
                    ln_gamma, ln_beta, W1, b1, W2, b2)
    sid_rows = seq_id_table[1:S + 1]
    raw, mout = _sc_pack(gid, maskI, event, pos_table, sid_rows,
                         empty_tokens)
    states = raw.reshape(B, S, K, H)
    seq_mask = (mout != 0).reshape(B, S, K)
    return states, seq_mask


# speedup vs baseline: 1.0152x; 1.0152x over previous
import functools

import jax
import jax.numpy as jnp
from jax import lax
from jax.experimental import pallas as pl
from jax.experimental.pallas import tpu as pltpu
from jax.experimental.pallas import tpu_sc as plsc

B, L, K, H, S, V, TG = 8, 2048, 512, 128, 8, 100000, 64
NT = B * L
NPAIR = B * S
_LANES = 16

_NC = 2
_NS = 16
_NW = _NC * _NS


_GCH = 128


def _make_sc_gather(nt):
    tok_per_w = nt // _NW
    ngch = tok_per_w // _GCH

    def body(embed_hbm, hist_hbm, post_hbm, auth_hbm, act_hbm,
             x0, x1, x2, x3, idx_v, rows_v, sem):
        wid = lax.axis_index("s") * _NC + lax.axis_index("c")
        base = wid * tok_per_w
        srcs = ((hist_hbm, x0), (post_hbm, x1), (auth_hbm, x2), (act_hbm, x3))
        for idx_hbm, out_hbm in srcs:
            for c in range(ngch):
                off = base + c * _GCH
                pltpu.sync_copy(idx_hbm.at[pl.ds(off, _GCH)], idx_v)
                pltpu.async_copy(embed_hbm.at[idx_v], rows_v, sem).wait()
                pltpu.sync_copy(rows_v, out_hbm.at[pl.ds(off, _GCH)])

    mesh = plsc.VectorSubcoreMesh(core_axis_name="c", subcore_axis_name="s")
    xt = jax.ShapeDtypeStruct((nt, H), jnp.float32)
    return functools.partial(
        pl.kernel, mesh=mesh,
        out_type=[xt, xt, xt, xt],
        compiler_params=pltpu.CompilerParams(needs_layout_passes=False),
        scratch_types=[
            pltpu.VMEM((_GCH,), jnp.int32),
            pltpu.VMEM((_GCH, H), jnp.float32),
            pltpu.SemaphoreType.DMA,
        ],
    )(body)


_BT = 4096


def _tc_mlp_body(x0, x1, x2, x3, gap, tgp, gamma, beta, w1, b1, w2, b2, out):
    oh = jnp.where(gap[...] == lax.broadcasted_iota(jnp.int32, (_BT, H), 1),
                   1.0, 0.0)
    x4 = jnp.dot(oh, tgp[...], preferred_element_type=jnp.float32,
                 precision=lax.Precision.HIGHEST)
    x = jnp.concatenate(
        [x0[...], x1[...], x2[...], x3[...], x4], axis=1)
    mu = jnp.mean(x, axis=-1, keepdims=True)
    var = jnp.mean((x - mu) ** 2, axis=-1, keepdims=True)
    xn = (x - mu) * lax.rsqrt(var + 1e-5) * gamma[...] + beta[...]
    h1 = jnp.dot(xn, w1[...], preferred_element_type=jnp.float32) + b1[...]
    h1 = h1 * jax.nn.sigmoid(h1)
    ev = jnp.dot(h1, w2[...], preferred_element_type=jnp.float32) + b2[...]
    out[...] = ev


def _tc_mlp(xs, gap2d, tgpad, ln_gamma, ln_beta, W1, b1, W2, b2, nt=NT):
    D5 = 5 * H
    D4 = 4 * H
    grid = (nt // _BT,)
    xspec = pl.BlockSpec((_BT, H), lambda i: (i, 0))

    def full(shape):
        return pl.BlockSpec(shape, lambda i: tuple(0 for _ in shape))

    return pl.pallas_call(
        _tc_mlp_body,
        grid=grid,
        in_specs=[xspec] * 4 + [pl.BlockSpec((_BT, 1), lambda i: (i, 0)),
                                full((H, H)), full((1, D5)), full((1, D5)),
                                full((D5, D4)), full((1, D4)),
                                full((D4, H)), full((1, H))],
        out_specs=pl.BlockSpec((_BT, H), lambda i: (i, 0)),
        out_shape=jax.ShapeDtypeStruct((nt, H), jnp.float32),
    )(*xs, gap2d, tgpad, ln_gamma.reshape(1, D5), ln_beta.reshape(1, D5),
      W1, b1.reshape(1, D4), W2, b2.reshape(1, H))


_PCH = 128
_NPCH = K // _PCH
_NVL = L // _LANES
_PAIRS_PER_W = NPAIR // _NW


def _sc_pack_body(gid_hbm, msk_hbm, event_hbm, pos_hbm, sid_hbm, emp_hbm,
                  states_hbm, mout_hbm,
                  ids_v, msk_v, idxl_v, chunk_v, pos_v, zbuf_v, sid_v,
                  emp_v, emp2_v, pos0_v, mko_v, sem):
    wid = lax.axis_index("s") * _NC + lax.axis_index("c")
    iota = lax.iota(jnp.int32, _LANES)
    zero16 = jnp.zeros((_LANES,), jnp.int32)
    fzero16 = jnp.zeros((_LANES,), jnp.float32)

    def zb_body(r, _):
        for j2 in range(H // _LANES):
            zbuf_v[r, pl.ds(j2 * _LANES, _LANES)] = fzero16
        return 0

    lax.fori_loop(0, _PCH, zb_body, 0)
    pltpu.sync_copy(pos_hbm.at[0], pos0_v)

    for p in range(_PAIRS_PER_W):
        pair = wid + p * _NW
        b = pair // S
        s = pair % S
        pltpu.sync_copy(gid_hbm.at[b], ids_v)
        pltpu.sync_copy(msk_hbm.at[b], msk_v)
        pltpu.sync_copy(sid_hbm.at[s], sid_v)
        pltpu.sync_copy(emp_hbm.at[s], emp_v)

        sval = s + 1

        def count_body(j, c):
            ids16 = ids_v[pl.ds(j * _LANES, _LANES)]
            mk16 = msk_v[pl.ds(j * _LANES, _LANES)]
            m = (ids16 == sval) & (mk16 != 0)
            return c + jnp.sum(jnp.where(m, 1, 0))

        count = lax.fori_loop(0, _NVL, count_body, jnp.int32(0))
        start = jnp.maximum(count - K, 0)
        n = count - start

        def zidx_body(j, _):
            idxl_v[pl.ds(j * _LANES, _LANES)] = j * _LANES + iota
            return 0

        lax.fori_loop(0, K // _LANES, zidx_body, 0)

        def rank_body(j, c):
            ids16 = ids_v[pl.ds(j * _LANES, _LANES)]
            mk16 = msk_v[pl.ds(j * _LANES, _LANES)]
            m = (ids16 == sval) & (mk16 != 0)
            mi = jnp.where(m, 1, 0)
            rank = plsc.cumsum(mi) + c - 1
            slot = rank - start
            wm = m & (slot >= 0)
            slot_c = jnp.maximum(slot, 0)
            gidx = b * L + j * _LANES + iota
            plsc.store_scatter(idxl_v, [slot_c], gidx, mask=wm)
            return c + jnp.sum(mi)

        lax.fori_loop(0, _NVL, rank_body, jnp.int32(0))

        for c4 in range(_NPCH):
            k0 = c4 * _PCH

            @pl.when(k0 < n)
            def _(c4=c4, k0=k0):
                pltpu.async_copy(event_hbm.at[idxl_v.at[pl.ds(k0, _PCH)]],
                                 chunk_v.at[c4], sem)

        for c4 in range(_NPCH):
            k0 = c4 * _PCH
            live = n - k0
            dst = states_hbm.at[pl.ds(pair * K + k0, _PCH)]

            def live_path(c4=c4, k0=k0, live=live, dst=dst):
                pltpu.make_async_copy(
                    event_hbm.at[idxl_v.at[pl.ds(k0, _PCH)]],
                    chunk_v.at[c4], sem).wait()
                pltpu.sync_copy(pos_hbm.at[pl.ds(k0, _PCH)], pos_v)

                def row_fn(r, _):
                    multf = jnp.where(r < live, 1.0, 0.0)
                    for j2 in range(H // _LANES):
                        sl = pl.ds(j2 * _LANES, _LANES)
                        chunk_v[c4, r, sl] = (chunk_v[c4, r, sl]
                                              + pos_v[r, sl]
                                              + sid_v[sl]) * multf
                    return 0

                lax.fori_loop(0, _PCH, row_fn, 0)
                pltpu.sync_copy(chunk_v.at[c4], dst)

            if c4 == 0:
                @pl.when(n > 0)
                def _(live_path=live_path):
                    live_path()

                @pl.when(n == 0)
                def _(dst=dst):
                    pltpu.sync_copy(zbuf_v, dst)
                    for j2 in range(H // _LANES):
                        sl = pl.ds(j2 * _LANES, _LANES)
                        emp2_v[0, sl] = emp_v[sl] + pos0_v[sl] + sid_v[sl]
                    pltpu.sync_copy(emp2_v,
                                    states_hbm.at[pl.ds(pair * K, 1)])
            else:
                @pl.when(live > 0)
                def _(live_path=live_path):
                    live_path()

                @pl.when(live <= 0)
                def _(dst=dst):
                    pltpu.sync_copy(zbuf_v, dst)

        def mk_body(j, _):
            k16 = j * _LANES + iota
            mv = (k16 < n) | ((k16 == 0) & (n == 0))
            mko_v[pl.ds(j * _LANES, _LANES)] = jnp.where(mv, 1, 0)
            return 0

        lax.fori_loop(0, K // _LANES, mk_body, 0)
        pltpu.sync_copy(mko_v, mout_hbm.at[pair])


def _sc_pack(gid, maskI, event, pos_table, sid_rows, empty_tokens):
    mesh = plsc.VectorSubcoreMesh(core_axis_name="c", subcore_axis_name="s")
    fn = functools.partial(
        pl.kernel, mesh=mesh,
        out_type=[jax.ShapeDtypeStruct((NPAIR * K, H), jnp.float32),
                  jax.ShapeDtypeStruct((NPAIR, K), jnp.int32)],
        compiler_params=pltpu.CompilerParams(needs_layout_passes=False),
        scratch_types=[
            pltpu.VMEM((L,), jnp.int32),
            pltpu.VMEM((L,), jnp.int32),
            pltpu.VMEM((K,), jnp.int32),
            pltpu.VMEM((_NPCH, _PCH, H), jnp.float32),
            pltpu.VMEM((_PCH, H), jnp.float32),
            pltpu.VMEM((_PCH, H), jnp.float32),
            pltpu.VMEM((H,), jnp.float32),
            pltpu.VMEM((H,), jnp.float32),
            pltpu.VMEM((1, H), jnp.float32),
            pltpu.VMEM((H,), jnp.float32),
            pltpu.VMEM((K,), jnp.int32),
            pltpu.SemaphoreType.DMA,
        ],
    )(_sc_pack_body)
    return fn(gid, maskI, event, pos_table, sid_rows, empty_tokens)


def kernel(history_tokens, history_post_tokens, history_author_tokens,
           history_action_tokens, history_time_gap, history_group_ids,
           history_mask, embed_table, time_gap_table, seq_id_table, pos_table,
           ln_gamma, ln_beta, W1, b1, W2, b2, empty_tokens):
    hist = history_tokens.reshape(NT).astype(jnp.int32)
    post = history_post_tokens.reshape(NT).astype(jnp.int32)
    auth = history_author_tokens.reshape(NT).astype(jnp.int32)
    act = history_action_tokens.reshape(NT).astype(jnp.int32)
    gap = history_time_gap.reshape(NT).astype(jnp.int32)
    gid = history_group_ids.astype(jnp.int32)
    maskI = history_mask.astype(jnp.int32)

    tgpad = jnp.zeros((H, H), jnp.float32).at[:TG + 1].set(time_gap_table)
    xs = _make_sc_gather(NT)(embed_table, hist, post, auth, act)
    event = _tc_mlp(xs, gap.reshape(NT, 1), tgpad,
                    ln_gamma, ln_beta, W1, b1, W2, b2)
    sid_rows = seq_id_table[1:S + 1]
    raw, mout = _sc_pack(gid, maskI, event, pos_table, sid_rows,
                         empty_tokens)
    states = raw.reshape(B, S, K, H)
    seq_mask = (mout != 0).reshape(B, S, K)
    return states, seq_mask


# speedup vs baseline: 1.0378x; 1.0222x over previous
import functools

import jax
import jax.numpy as jnp
from jax import lax
from jax.experimental import pallas as pl
from jax.experimental.pallas import tpu as pltpu
from jax.experimental.pallas import tpu_sc as plsc

B, L, K, H, S, V, TG = 8, 2048, 512, 128, 8, 100000, 64
NT = B * L
NPAIR = B * S
_LANES = 16

_NC = 2
_NS = 16
_NW = _NC * _NS


_GCH = 128


def _make_sc_gather(nt):
    tok_per_w = nt // _NW
    ngch = tok_per_w // _GCH

    def body(embed_hbm, hist_hbm, post_hbm, auth_hbm, act_hbm,
             x0, x1, x2, x3, idx_v, rows_v, sem):
        wid = lax.axis_index("s") * _NC + lax.axis_index("c")
        base = wid * tok_per_w
        srcs = ((hist_hbm, x0), (post_hbm, x1), (auth_hbm, x2), (act_hbm, x3))
        for idx_hbm, out_hbm in srcs:
            for c in range(ngch):
                off = base + c * _GCH
                pltpu.sync_copy(idx_hbm.at[pl.ds(off, _GCH)], idx_v)
                pltpu.async_copy(embed_hbm.at[idx_v], rows_v, sem).wait()
                pltpu.sync_copy(rows_v, out_hbm.at[pl.ds(off, _GCH)])

    mesh = plsc.VectorSubcoreMesh(core_axis_name="c", subcore_axis_name="s")
    xt = jax.ShapeDtypeStruct((nt, H), jnp.float32)
    return functools.partial(
        pl.kernel, mesh=mesh,
        out_type=[xt, xt, xt, xt],
        compiler_params=pltpu.CompilerParams(needs_layout_passes=False),
        scratch_types=[
            pltpu.VMEM((_GCH,), jnp.int32),
            pltpu.VMEM((_GCH, H), jnp.float32),
            pltpu.SemaphoreType.DMA,
        ],
    )(body)


_BT = 2048


def _tc_mlp_body(x0, x1, x2, x3, gap, tgp, gamma, beta, w1, b1, w2, b2, out):
    oh = jnp.where(gap[...] == lax.broadcasted_iota(jnp.int32, (_BT, H), 1),
                   1.0, 0.0)
    x4 = jnp.dot(oh, tgp[...], preferred_element_type=jnp.float32,
                 precision=lax.Precision.HIGHEST)
    x = jnp.concatenate(
        [x0[...], x1[...], x2[...], x3[...], x4], axis=1)
    mu = jnp.mean(x, axis=-1, keepdims=True)
    var = jnp.mean((x - mu) ** 2, axis=-1, keepdims=True)
    xn = (x - mu) * lax.rsqrt(var + 1e-5) * gamma[...] + beta[...]
    h1 = jnp.dot(xn, w1[...], preferred_element_type=jnp.float32) + b1[...]
    h1 = h1 * jax.nn.sigmoid(h1)
    ev = jnp.dot(h1, w2[...], preferred_element_type=jnp.float32) + b2[...]
    out[...] = ev


def _tc_mlp(xs, gap2d, tgpad, ln_gamma, ln_beta, W1, b1, W2, b2, nt=NT):
    D5 = 5 * H
    D4 = 4 * H
    grid = (nt // _BT,)
    xspec = pl.BlockSpec((_BT, H), lambda i: (i, 0))

    def full(shape):
        return pl.BlockSpec(shape, lambda i: tuple(0 for _ in shape))

    return pl.pallas_call(
        _tc_mlp_body,
        grid=grid,
        in_specs=[xspec] * 4 + [pl.BlockSpec((_BT, 1), lambda i: (i, 0)),
                                full((H, H)), full((1, D5)), full((1, D5)),
                                full((D5, D4)), full((1, D4)),
                                full((D4, H)), full((1, H))],
        out_specs=pl.BlockSpec((_BT, H), lambda i: (i, 0)),
        out_shape=jax.ShapeDtypeStruct((nt, H), jnp.float32),
    )(*xs, gap2d, tgpad, ln_gamma.reshape(1, D5), ln_beta.reshape(1, D5),
      W1, b1.reshape(1, D4), W2, b2.reshape(1, H))


_PCH = 128
_NPCH = K // _PCH
_NVL = L // _LANES
_PAIRS_PER_W = NPAIR // _NW


def _sc_pack_body(gid_hbm, msk_hbm, event_hbm, emp_hbm,
                  states_hbm, mout_hbm,
                  ids_v, msk_v, idxl_v, chunk_v, emp_v, mko_v, sem):
    wid = lax.axis_index("s") * _NC + lax.axis_index("c")
    iota = lax.iota(jnp.int32, _LANES)
    zero16 = jnp.zeros((_LANES,), jnp.int32)

    for p in range(_PAIRS_PER_W):
        pair = wid + p * _NW
        b = pair // S
        s = pair % S
        pltpu.sync_copy(gid_hbm.at[b], ids_v)
        pltpu.sync_copy(msk_hbm.at[b], msk_v)
        pltpu.sync_copy(emp_hbm.at[s], emp_v)

        sval = s + 1

        def count_body(j, c):
            ids16 = ids_v[pl.ds(j * _LANES, _LANES)]
            mk16 = msk_v[pl.ds(j * _LANES, _LANES)]
            m = (ids16 == sval) & (mk16 != 0)
            return c + jnp.sum(jnp.where(m, 1, 0))

        count = lax.fori_loop(0, _NVL, count_body, jnp.int32(0))
        start = jnp.maximum(count - K, 0)
        n = count - start

        def zidx_body(j, _):
            idxl_v[pl.ds(j * _LANES, _LANES)] = j * _LANES + iota
            return 0

        lax.fori_loop(0, K // _LANES, zidx_body, 0)

        def rank_body(j, c):
            ids16 = ids_v[pl.ds(j * _LANES, _LANES)]
            mk16 = msk_v[pl.ds(j * _LANES, _LANES)]
            m = (ids16 == sval) & (mk16 != 0)
            mi = jnp.where(m, 1, 0)
            rank = plsc.cumsum(mi) + c - 1
            slot = rank - start
            wm = m & (slot >= 0)
            slot_c = jnp.maximum(slot, 0)
            gidx = b * L + j * _LANES + iota
            plsc.store_scatter(idxl_v, [slot_c], gidx, mask=wm)
            return c + jnp.sum(mi)

        lax.fori_loop(0, _NVL, rank_body, jnp.int32(0))

        for c4 in range(_NPCH):
            k0 = c4 * _PCH

            @pl.when(k0 < n)
            def _(c4=c4, k0=k0):
                pltpu.async_copy(event_hbm.at[idxl_v.at[pl.ds(k0, _PCH)]],
                                 chunk_v.at[c4], sem)

        for c4 in range(_NPCH):
            k0 = c4 * _PCH

            @pl.when(k0 < n)
            def _(c4=c4, k0=k0):
                pltpu.make_async_copy(
                    event_hbm.at[idxl_v.at[pl.ds(k0, _PCH)]],
                    chunk_v.at[c4], sem).wait()

            if c4 == 0:
                @pl.when(n == 0)
                def _():
                    for j2 in range(H // _LANES):
                        sl = pl.ds(j2 * _LANES, _LANES)
                        chunk_v[0, 0, sl] = emp_v[sl]

                pltpu.sync_copy(chunk_v.at[0],
                                states_hbm.at[pl.ds(pair * K, _PCH)])
            else:
                @pl.when(k0 < n)
                def _(c4=c4, k0=k0):
                    pltpu.sync_copy(chunk_v.at[c4],
                                    states_hbm.at[pl.ds(pair * K + k0, _PCH)])

        def mk_body(j, _):
            k16 = j * _LANES + iota
            mv = (k16 < n) | ((k16 == 0) & (n == 0))
            mko_v[pl.ds(j * _LANES, _LANES)] = jnp.where(mv, 1, 0)
            return 0

        lax.fori_loop(0, K // _LANES, mk_body, 0)
        pltpu.sync_copy(mko_v, mout_hbm.at[pair])


def _sc_pack(gid, maskI, event, empty_tokens):
    mesh = plsc.VectorSubcoreMesh(core_axis_name="c", subcore_axis_name="s")
    fn = functools.partial(
        pl.kernel, mesh=mesh,
        out_type=[jax.ShapeDtypeStruct((NPAIR * K, H), jnp.float32),
                  jax.ShapeDtypeStruct((NPAIR, K), jnp.int32)],
        compiler_params=pltpu.CompilerParams(needs_layout_passes=False),
        scratch_types=[
            pltpu.VMEM((L,), jnp.int32),
            pltpu.VMEM((L,), jnp.int32),
            pltpu.VMEM((K,), jnp.int32),
            pltpu.VMEM((_NPCH, _PCH, H), jnp.float32),
            pltpu.VMEM((H,), jnp.float32),
            pltpu.VMEM((K,), jnp.int32),
            pltpu.SemaphoreType.DMA,
        ],
    )(_sc_pack_body)
    return fn(gid, maskI, event, empty_tokens)


def _tc_final_body(raw, maskf, pos, sid, out):
    x = raw[0]
    m = maskf[0]
    val = x + pos[...][None, :, :] + sid[...][:, None, :]
    out[0] = jnp.where(m > 0.0, val, 0.0)


def _tc_final(raw, maskf, pos_table, sid_rows):
    return pl.pallas_call(
        _tc_final_body,
        grid=(B,),
        in_specs=[
            pl.BlockSpec((1, S, K, H), lambda i: (i, 0, 0, 0)),
            pl.BlockSpec((1, S, K, 1), lambda i: (i, 0, 0, 0)),
            pl.BlockSpec((K, H), lambda i: (0, 0)),
            pl.BlockSpec((S, H), lambda i: (0, 0)),
        ],
        out_specs=pl.BlockSpec((1, S, K, H), lambda i: (i, 0, 0, 0)),
        out_shape=jax.ShapeDtypeStruct((B, S, K, H), jnp.float32),
    )(raw, maskf, pos_table, sid_rows)


def kernel(history_tokens, history_post_tokens, history_author_tokens,
           history_action_tokens, history_time_gap, history_group_ids,
           history_mask, embed_table, time_gap_table, seq_id_table, pos_table,
           ln_gamma, ln_beta, W1, b1, W2, b2, empty_tokens):
    hist = history_tokens.reshape(NT).astype(jnp.int32)
    post = history_post_tokens.reshape(NT).astype(jnp.int32)
    auth = history_author_tokens.reshape(NT).astype(jnp.int32)
    act = history_action_tokens.reshape(NT).astype(jnp.int32)
    gap = history_time_gap.reshape(NT).astype(jnp.int32)
    gid = history_group_ids.astype(jnp.int32)
    maskI = history_mask.astype(jnp.int32)

    tgpad = jnp.zeros((H, H), jnp.float32).at[:TG + 1].set(time_gap_table)
    xs = _make_sc_gather(NT)(embed_table, hist, post, auth, act)
    event = _tc_mlp(xs, gap.reshape(NT, 1), tgpad,
                    ln_gamma, ln_beta, W1, b1, W2, b2)
    sid_rows = seq_id_table[1:S + 1]
    raw, mout = _sc_pack(gid, maskI, event, empty_tokens)
    maskf = mout.astype(jnp.float32).reshape(B, S, K, 1)
    states = _tc_final(raw.reshape(B, S, K, H), maskf, pos_table, sid_rows)
    seq_mask = (mout != 0).reshape(B, S, K)
    return states, seq_mask


# speedup vs baseline: 1.1582x; 1.1160x over previous
import functools

import jax
import jax.numpy as jnp
from jax import lax
from jax.experimental import pallas as pl
from jax.experimental.pallas import tpu as pltpu
from jax.experimental.pallas import tpu_sc as plsc

B, L, K, H, S, V, TG = 8, 2048, 512, 128, 8, 100000, 64
NT = B * L
NPAIR = B * S
_LANES = 16

_NC = 2
_NS = 16
_NW = _NC * _NS


_GCH = 128


def _make_sc_gather(nt):
    tok_per_w = nt // _NW
    ngch = tok_per_w // _GCH

    def body(embed_hbm, hist_hbm, post_hbm, auth_hbm, act_hbm,
             x0, x1, x2, x3, idx_v, rows_v, sem0, sem1):
        sems = (sem0, sem1)
        wid = lax.axis_index("s") * _NC + lax.axis_index("c")
        base = wid * tok_per_w
        srcs = ((hist_hbm, x0), (post_hbm, x1), (auth_hbm, x2), (act_hbm, x3))
        for t, (idx_hbm, _) in enumerate(srcs):
            pltpu.sync_copy(idx_hbm.at[pl.ds(base, tok_per_w)], idx_v.at[t])
        jobs = [(t, c) for t in range(len(srcs)) for c in range(ngch)]

        def fire(i):
            t, c = jobs[i]
            pltpu.async_copy(
                embed_hbm.at[idx_v.at[t, pl.ds(c * _GCH, _GCH)]],
                rows_v.at[i % 2], sems[i % 2])

        def drain(i):
            t, c = jobs[i]
            pltpu.make_async_copy(
                embed_hbm.at[idx_v.at[t, pl.ds(c * _GCH, _GCH)]],
                rows_v.at[i % 2], sems[i % 2]).wait()
            pltpu.sync_copy(rows_v.at[i % 2],
                            srcs[t][1].at[pl.ds(base + c * _GCH, _GCH)])

        fire(0)
        for i in range(len(jobs)):
            if i + 1 < len(jobs):
                fire(i + 1)
            drain(i)

    mesh = plsc.VectorSubcoreMesh(core_axis_name="c", subcore_axis_name="s")
    xt = jax.ShapeDtypeStruct((nt, H), jnp.float32)
    return functools.partial(
        pl.kernel, mesh=mesh,
        out_type=[xt, xt, xt, xt],
        compiler_params=pltpu.CompilerParams(needs_layout_passes=False),
        scratch_types=[
            pltpu.VMEM((4, tok_per_w), jnp.int32),
            pltpu.VMEM((2, _GCH, H), jnp.float32),
            pltpu.SemaphoreType.DMA,
            pltpu.SemaphoreType.DMA,
        ],
    )(body)


_BT = 2048


def _tc_mlp_body(x0, x1, x2, x3, gap, tgp, gamma, beta, w1, b1, w2, b2, out):
    oh = jnp.where(gap[...] == lax.broadcasted_iota(jnp.int32, (_BT, H), 1),
                   1.0, 0.0)
    x4 = jnp.dot(oh, tgp[...], preferred_element_type=jnp.float32,
                 precision=lax.Precision.HIGHEST)
    x = jnp.concatenate(
        [x0[...], x1[...], x2[...], x3[...], x4], axis=1)
    mu = jnp.mean(x, axis=-1, keepdims=True)
    var = jnp.mean((x - mu) ** 2, axis=-1, keepdims=True)
    xn = (x - mu) * lax.rsqrt(var + 1e-5) * gamma[...] + beta[...]
    h1 = jnp.dot(xn, w1[...], preferred_element_type=jnp.float32) + b1[...]
    h1 = h1 * jax.nn.sigmoid(h1)
    ev = jnp.dot(h1, w2[...], preferred_element_type=jnp.float32) + b2[...]
    out[...] = ev


def _tc_mlp(xs, gap2d, tgpad, ln_gamma, ln_beta, W1, b1, W2, b2, nt=NT):
    D5 = 5 * H
    D4 = 4 * H
    grid = (nt // _BT,)
    xspec = pl.BlockSpec((_BT, H), lambda i: (i, 0))

    def full(shape):
        return pl.BlockSpec(shape, lambda i: tuple(0 for _ in shape))

    return pl.pallas_call(
        _tc_mlp_body,
        grid=grid,
        in_specs=[xspec] * 4 + [pl.BlockSpec((_BT, 1), lambda i: (i, 0)),
                                full((H, H)), full((1, D5)), full((1, D5)),
                                full((D5, D4)), full((1, D4)),
                                full((D4, H)), full((1, H))],
        out_specs=pl.BlockSpec((_BT, H), lambda i: (i, 0)),
        out_shape=jax.ShapeDtypeStruct((nt, H), jnp.float32),
    )(*xs, gap2d, tgpad, ln_gamma.reshape(1, D5), ln_beta.reshape(1, D5),
      W1, b1.reshape(1, D4), W2, b2.reshape(1, H))


_PCH = 128
_NPCH = K // _PCH
_NVL = L // _LANES
_PAIRS_PER_W = NPAIR // _NW


def _sc_pack_body(gid_hbm, msk_hbm, event_hbm, emp_hbm,
                  states_hbm, mout_hbm,
                  ids_v, msk_v, idxl_v, chunk_v, emp_v, mko_v, sem):
    wid = lax.axis_index("s") * _NC + lax.axis_index("c")
    iota = lax.iota(jnp.int32, _LANES)
    zero16 = jnp.zeros((_LANES,), jnp.int32)

    for p in range(_PAIRS_PER_W):
        pair = wid + p * _NW
        b = pair // S
        s = pair % S
        pltpu.sync_copy(gid_hbm.at[b], ids_v)
        pltpu.sync_copy(msk_hbm.at[b], msk_v)
        pltpu.sync_copy(emp_hbm.at[s], emp_v)

        sval = s + 1

        def count_body(j, c):
            ids16 = ids_v[pl.ds(j * _LANES, _LANES)]
            mk16 = msk_v[pl.ds(j * _LANES, _LANES)]
            m = (ids16 == sval) & (mk16 != 0)
            return c + jnp.sum(jnp.where(m, 1, 0))

        count = lax.fori_loop(0, _NVL, count_body, jnp.int32(0))
        start = jnp.maximum(count - K, 0)
        n = count - start

        def zidx_body(j, _):
            idxl_v[pl.ds(j * _LANES, _LANES)] = j * _LANES + iota
            return 0

        lax.fori_loop(0, K // _LANES, zidx_body, 0)

        def rank_body(j, c):
            ids16 = ids_v[pl.ds(j * _LANES, _LANES)]
            mk16 = msk_v[pl.ds(j * _LANES, _LANES)]
            m = (ids16 == sval) & (mk16 != 0)
            mi = jnp.where(m, 1, 0)
            rank = plsc.cumsum(mi) + c - 1
            slot = rank - start
            wm = m & (slot >= 0)
            slot_c = jnp.maximum(slot, 0)
            gidx = b * L + j * _LANES + iota
            plsc.store_scatter(idxl_v, [slot_c], gidx, mask=wm)
            return c + jnp.sum(mi)

        lax.fori_loop(0, _NVL, rank_body, jnp.int32(0))

        for c4 in range(_NPCH):
            k0 = c4 * _PCH

            @pl.when(k0 < n)
            def _(c4=c4, k0=k0):
                pltpu.async_copy(event_hbm.at[idxl_v.at[pl.ds(k0, _PCH)]],
                                 chunk_v.at[c4], sem)

        for c4 in range(_NPCH):
            k0 = c4 * _PCH

            @pl.when(k0 < n)
            def _(c4=c4, k0=k0):
                pltpu.make_async_copy(
                    event_hbm.at[idxl_v.at[pl.ds(k0, _PCH)]],
                    chunk_v.at[c4], sem).wait()

            if c4 == 0:
                @pl.when(n == 0)
                def _():
                    for j2 in range(H // _LANES):
                        sl = pl.ds(j2 * _LANES, _LANES)
                        chunk_v[0, 0, sl] = emp_v[sl]

                pltpu.sync_copy(chunk_v.at[0],
                                states_hbm.at[pl.ds(pair * K, _PCH)])
            else:
                @pl.when(k0 < n)
                def _(c4=c4, k0=k0):
                    pltpu.sync_copy(chunk_v.at[c4],
                                    states_hbm.at[pl.ds(pair * K + k0, _PCH)])

        def mk_body(j, _):
            k16 = j * _LANES + iota
            mv = (k16 < n) | ((k16 == 0) & (n == 0))
            mko_v[pl.ds(j * _LANES, _LANES)] = jnp.where(mv, 1, 0)
            return 0

        lax.fori_loop(0, K // _LANES, mk_body, 0)
        pltpu.sync_copy(mko_v, mout_hbm.at[pair])


def _sc_pack(gid, maskI, event, empty_tokens):
    mesh = plsc.VectorSubcoreMesh(core_axis_name="c", subcore_axis_name="s")
    fn = functools.partial(
        pl.kernel, mesh=mesh,
        out_type=[jax.ShapeDtypeStruct((NPAIR * K, H), jnp.float32),
                  jax.ShapeDtypeStruct((NPAIR, K), jnp.int32)],
        compiler_params=pltpu.CompilerParams(needs_layout_passes=False),
        scratch_types=[
            pltpu.VMEM((L,), jnp.int32),
            pltpu.VMEM((L,), jnp.int32),
            pltpu.VMEM((K,), jnp.int32),
            pltpu.VMEM((_NPCH, _PCH, H), jnp.float32),
            pltpu.VMEM((H,), jnp.float32),
            pltpu.VMEM((K,), jnp.int32),
            pltpu.SemaphoreType.DMA,
        ],
    )(_sc_pack_body)
    return fn(gid, maskI, event, empty_tokens)


def _tc_final_body(raw, maskf, pos, sid, out):
    x = raw[0]
    m = maskf[0]
    val = x + pos[...][None, :, :] + sid[...][:, None, :]
    out[0] = jnp.where(m > 0.0, val, 0.0)


def _tc_final(raw, maskf, pos_table, sid_rows):
    return pl.pallas_call(
        _tc_final_body,
        grid=(B,),
        in_specs=[
            pl.BlockSpec((1, S, K, H), lambda i: (i, 0, 0, 0)),
            pl.BlockSpec((1, S, K, 1), lambda i: (i, 0, 0, 0)),
            pl.BlockSpec((K, H), lambda i: (0, 0)),
            pl.BlockSpec((S, H), lambda i: (0, 0)),
        ],
        out_specs=pl.BlockSpec((1, S, K, H), lambda i: (i, 0, 0, 0)),
        out_shape=jax.ShapeDtypeStruct((B, S, K, H), jnp.float32),
    )(raw, maskf, pos_table, sid_rows)


def kernel(history_tokens, history_post_tokens, history_author_tokens,
           history_action_tokens, history_time_gap, history_group_ids,
           history_mask, embed_table, time_gap_table, seq_id_table, pos_table,
           ln_gamma, ln_beta, W1, b1, W2, b2, empty_tokens):
    hist = history_tokens.reshape(NT).astype(jnp.int32)
    post = history_post_tokens.reshape(NT).astype(jnp.int32)
    auth = history_author_tokens.reshape(NT).astype(jnp.int32)
    act = history_action_tokens.reshape(NT).astype(jnp.int32)
    gap = history_time_gap.reshape(NT).astype(jnp.int32)
    gid = history_group_ids.astype(jnp.int32)
    maskI = history_mask.astype(jnp.int32)

    tgpad = jnp.zeros((H, H), jnp.float32).at[:TG + 1].set(time_gap_table)
    xs = _make_sc_gather(NT)(embed_table, hist, post, auth, act)
    event = _tc_mlp(xs, gap.reshape(NT, 1), tgpad,
                    ln_gamma, ln_beta, W1, b1, W2, b2)
    sid_rows = seq_id_table[1:S + 1]
    raw, mout = _sc_pack(gid, maskI, event, empty_tokens)
    maskf = mout.astype(jnp.float32).reshape(B, S, K, 1)
    states = _tc_final(raw.reshape(B, S, K, H), maskf, pos_table, sid_rows)
    seq_mask = (mout != 0).reshape(B, S, K)
    return states, seq_mask


# speedup vs baseline: 1.1607x; 1.0022x over previous
import functools

import jax
import jax.numpy as jnp
from jax import lax
from jax.experimental import pallas as pl
from jax.experimental.pallas import tpu as pltpu
from jax.experimental.pallas import tpu_sc as plsc

B, L, K, H, S, V, TG = 8, 2048, 512, 128, 8, 100000, 64
NT = B * L
NPAIR = B * S
_LANES = 16

_NC = 2
_NS = 16
_NW = _NC * _NS


_GCH = 128


def _make_sc_gather(nt):
    tok_per_w = nt // _NW
    ngch = tok_per_w // _GCH

    def body(embed_hbm, hist_hbm, post_hbm, auth_hbm, act_hbm,
             x0, x1, x2, x3, idx_v, rows_v, sem0, sem1, sem2, sem3):
        sems = (sem0, sem1, sem2, sem3)
        nbuf = len(sems)
        wid = lax.axis_index("s") * _NC + lax.axis_index("c")
        base = wid * tok_per_w
        srcs = ((hist_hbm, x0), (post_hbm, x1), (auth_hbm, x2), (act_hbm, x3))
        for t, (idx_hbm, _) in enumerate(srcs):
            pltpu.sync_copy(idx_hbm.at[pl.ds(base, tok_per_w)], idx_v.at[t])
        jobs = [(t, c) for t in range(len(srcs)) for c in range(ngch)]

        def fire(i):
            t, c = jobs[i]
            pltpu.async_copy(
                embed_hbm.at[idx_v.at[t, pl.ds(c * _GCH, _GCH)]],
                rows_v.at[i % nbuf], sems[i % nbuf])

        def drain(i):
            t, c = jobs[i]
            pltpu.make_async_copy(
                embed_hbm.at[idx_v.at[t, pl.ds(c * _GCH, _GCH)]],
                rows_v.at[i % nbuf], sems[i % nbuf]).wait()
            pltpu.sync_copy(rows_v.at[i % nbuf],
                            srcs[t][1].at[pl.ds(base + c * _GCH, _GCH)])

        for i in range(nbuf - 1):
            fire(i)
        for i in range(len(jobs)):
            if i + nbuf - 1 < len(jobs):
                fire(i + nbuf - 1)
            drain(i)

    mesh = plsc.VectorSubcoreMesh(core_axis_name="c", subcore_axis_name="s")
    xt = jax.ShapeDtypeStruct((nt, H), jnp.float32)
    return functools.partial(
        pl.kernel, mesh=mesh,
        out_type=[xt, xt, xt, xt],
        compiler_params=pltpu.CompilerParams(needs_layout_passes=False),
        scratch_types=[
            pltpu.VMEM((4, tok_per_w), jnp.int32),
            pltpu.VMEM((4, _GCH, H), jnp.float32),
            pltpu.SemaphoreType.DMA,
            pltpu.SemaphoreType.DMA,
            pltpu.SemaphoreType.DMA,
            pltpu.SemaphoreType.DMA,
        ],
    )(body)


_BT = 2048


def _tc_mlp_body(x0, x1, x2, x3, gap, tgp, gamma, beta, w1, b1, w2, b2, out):
    oh = jnp.where(gap[...] == lax.broadcasted_iota(jnp.int32, (_BT, H), 1),
                   1.0, 0.0)
    x4 = jnp.dot(oh, tgp[...], preferred_element_type=jnp.float32,
                 precision=lax.Precision.HIGHEST)
    x = jnp.concatenate(
        [x0[...], x1[...], x2[...], x3[...], x4], axis=1)
    mu = jnp.mean(x, axis=-1, keepdims=True)
    var = jnp.mean((x - mu) ** 2, axis=-1, keepdims=True)
    xn = (x - mu) * lax.rsqrt(var + 1e-5) * gamma[...] + beta[...]
    h1 = jnp.dot(xn, w1[...], preferred_element_type=jnp.float32) + b1[...]
    h1 = h1 * jax.nn.sigmoid(h1)
    ev = jnp.dot(h1, w2[...], preferred_element_type=jnp.float32) + b2[...]
    out[...] = ev


def _tc_mlp(xs, gap2d, tgpad, ln_gamma, ln_beta, W1, b1, W2, b2, nt=NT):
    D5 = 5 * H
    D4 = 4 * H
    grid = (nt // _BT,)
    xspec = pl.BlockSpec((_BT, H), lambda i: (i, 0))

    def full(shape):
        return pl.BlockSpec(shape, lambda i: tuple(0 for _ in shape))

    return pl.pallas_call(
        _tc_mlp_body,
        grid=grid,
        in_specs=[xspec] * 4 + [pl.BlockSpec((_BT, 1), lambda i: (i, 0)),
                                full((H, H)), full((1, D5)), full((1, D5)),
                                full((D5, D4)), full((1, D4)),
                                full((D4, H)), full((1, H))],
        out_specs=pl.BlockSpec((_BT, H), lambda i: (i, 0)),
        out_shape=jax.ShapeDtypeStruct((nt, H), jnp.float32),
    )(*xs, gap2d, tgpad, ln_gamma.reshape(1, D5), ln_beta.reshape(1, D5),
      W1, b1.reshape(1, D4), W2, b2.reshape(1, H))


_PCH = 128
_NPCH = K // _PCH
_NVL = L // _LANES
_PAIRS_PER_W = NPAIR // _NW


def _sc_pack_body(gid_hbm, msk_hbm, event_hbm, emp_hbm,
                  states_hbm, mout_hbm,
                  ids_v, msk_v, idxl_v, chunk_v, emp_v, mko_v, sem):
    wid = lax.axis_index("s") * _NC + lax.axis_index("c")
    iota = lax.iota(jnp.int32, _LANES)
    zero16 = jnp.zeros((_LANES,), jnp.int32)

    for p in range(_PAIRS_PER_W):
        pair = wid + p * _NW
        b = pair // S
        s = pair % S
        pltpu.sync_copy(gid_hbm.at[b], ids_v)
        pltpu.sync_copy(msk_hbm.at[b], msk_v)
        pltpu.sync_copy(emp_hbm.at[s], emp_v)

        sval = s + 1

        def count_body(j, c):
            ids16 = ids_v[pl.ds(j * _LANES, _LANES)]
            mk16 = msk_v[pl.ds(j * _LANES, _LANES)]
            m = (ids16 == sval) & (mk16 != 0)
            return c + jnp.sum(jnp.where(m, 1, 0))

        count = lax.fori_loop(0, _NVL, count_body, jnp.int32(0))
        start = jnp.maximum(count - K, 0)
        n = count - start

        def zidx_body(j, _):
            idxl_v[pl.ds(j * _LANES, _LANES)] = j * _LANES + iota
            return 0

        lax.fori_loop(0, K // _LANES, zidx_body, 0)

        def rank_body(j, c):
            ids16 = ids_v[pl.ds(j * _LANES, _LANES)]
            mk16 = msk_v[pl.ds(j * _LANES, _LANES)]
            m = (ids16 == sval) & (mk16 != 0)
            mi = jnp.where(m, 1, 0)
            rank = plsc.cumsum(mi) + c - 1
            slot = rank - start
            wm = m & (slot >= 0)
            slot_c = jnp.maximum(slot, 0)
            gidx = b * L + j * _LANES + iota
            plsc.store_scatter(idxl_v, [slot_c], gidx, mask=wm)
            return c + jnp.sum(mi)

        lax.fori_loop(0, _NVL, rank_body, jnp.int32(0))

        for c4 in range(_NPCH):
            k0 = c4 * _PCH

            @pl.when(k0 < n)
            def _(c4=c4, k0=k0):
                pltpu.async_copy(event_hbm.at[idxl_v.at[pl.ds(k0, _PCH)]],
                                 chunk_v.at[c4], sem)

        for c4 in range(_NPCH):
            k0 = c4 * _PCH

            @pl.when(k0 < n)
            def _(c4=c4, k0=k0):
                pltpu.make_async_copy(
                    event_hbm.at[idxl_v.at[pl.ds(k0, _PCH)]],
                    chunk_v.at[c4], sem).wait()

            if c4 == 0:
                @pl.when(n == 0)
                def _():
                    for j2 in range(H // _LANES):
                        sl = pl.ds(j2 * _LANES, _LANES)
                        chunk_v[0, 0, sl] = emp_v[sl]

                pltpu.sync_copy(chunk_v.at[0],
                                states_hbm.at[pl.ds(pair * K, _PCH)])
            else:
                @pl.when(k0 < n)
                def _(c4=c4, k0=k0):
                    pltpu.sync_copy(chunk_v.at[c4],
                                    states_hbm.at[pl.ds(pair * K + k0, _PCH)])

        def mk_body(j, _):
            k16 = j * _LANES + iota
            mv = (k16 < n) | ((k16 == 0) & (n == 0))
            mko_v[pl.ds(j * _LANES, _LANES)] = jnp.where(mv, 1, 0)
            return 0

        lax.fori_loop(0, K // _LANES, mk_body, 0)
        pltpu.sync_copy(mko_v, mout_hbm.at[pair])


def _sc_pack(gid, maskI, event, empty_tokens):
    mesh = plsc.VectorSubcoreMesh(core_axis_name="c", subcore_axis_name="s")
    fn = functools.partial(
        pl.kernel, mesh=mesh,
        out_type=[jax.ShapeDtypeStruct((NPAIR * K, H), jnp.float32),
                  jax.ShapeDtypeStruct((NPAIR, K), jnp.int32)],
        compiler_params=pltpu.CompilerParams(needs_layout_passes=False),
        scratch_types=[
            pltpu.VMEM((L,), jnp.int32),
            pltpu.VMEM((L,), jnp.int32),
            pltpu.VMEM((K,), jnp.int32),
            pltpu.VMEM((_NPCH, _PCH, H), jnp.float32),
            pltpu.VMEM((H,), jnp.float32),
            pltpu.VMEM((K,), jnp.int32),
            pltpu.SemaphoreType.DMA,
        ],
    )(_sc_pack_body)
    return fn(gid, maskI, event, empty_tokens)


def _tc_final_body(raw, maskf, pos, sid, out):
    x = raw[0]
    m = maskf[0]
    val = x + pos[...][None, :, :] + sid[...][:, None, :]
    out[0] = jnp.where(m > 0.0, val, 0.0)


def _tc_final(raw, maskf, pos_table, sid_rows):
    return pl.pallas_call(
        _tc_final_body,
        grid=(B,),
        in_specs=[
            pl.BlockSpec((1, S, K, H), lambda i: (i, 0, 0, 0)),
            pl.BlockSpec((1, S, K, 1), lambda i: (i, 0, 0, 0)),
            pl.BlockSpec((K, H), lambda i: (0, 0)),
            pl.BlockSpec((S, H), lambda i: (0, 0)),
        ],
        out_specs=pl.BlockSpec((1, S, K, H), lambda i: (i, 0, 0, 0)),
        out_shape=jax.ShapeDtypeStruct((B, S, K, H), jnp.float32),
    )(raw, maskf, pos_table, sid_rows)


def kernel(history_tokens, history_post_tokens, history_author_tokens,
           history_action_tokens, history_time_gap, history_group_ids,
           history_mask, embed_table, time_gap_table, seq_id_table, pos_table,
           ln_gamma, ln_beta, W1, b1, W2, b2, empty_tokens):
    hist = history_tokens.reshape(NT).astype(jnp.int32)
    post = history_post_tokens.reshape(NT).astype(jnp.int32)
    auth = history_author_tokens.reshape(NT).astype(jnp.int32)
    act = history_action_tokens.reshape(NT).astype(jnp.int32)
    gap = history_time_gap.reshape(NT).astype(jnp.int32)
    gid = history_group_ids.astype(jnp.int32)
    maskI = history_mask.astype(jnp.int32)

    tgpad = jnp.zeros((H, H), jnp.float32).at[:TG + 1].set(time_gap_table)
    xs = _make_sc_gather(NT)(embed_table, hist, post, auth, act)
    event = _tc_mlp(xs, gap.reshape(NT, 1), tgpad,
                    ln_gamma, ln_beta, W1, b1, W2, b2)
    sid_rows = seq_id_table[1:S + 1]
    raw, mout = _sc_pack(gid, maskI, event, empty_tokens)
    maskf = mout.astype(jnp.float32).reshape(B, S, K, 1)
    states = _tc_final(raw.reshape(B, S, K, H), maskf, pos_table, sid_rows)
    seq_mask = (mout != 0).reshape(B, S, K)
    return states, seq_mask


# speedup vs baseline: 1.1675x; 1.0059x over previous
import functools

import jax
import jax.numpy as jnp
from jax import lax
from jax.experimental import pallas as pl
from jax.experimental.pallas import tpu as pltpu
from jax.experimental.pallas import tpu_sc as plsc

B, L, K, H, S, V, TG = 8, 2048, 512, 128, 8, 100000, 64
NT = B * L
NPAIR = B * S
_LANES = 16

_NC = 2
_NS = 16
_NW = _NC * _NS


_GCH = 128


def _make_sc_gather(nt):
    tok_per_w = nt // _NW
    ngch = tok_per_w // _GCH

    def body(embed_hbm, hist_hbm, post_hbm, auth_hbm, act_hbm,
             x0, x1, x2, x3, idx_v, rows_v, sem0, sem1, sem2, sem3):
        sems = (sem0, sem1, sem2, sem3)
        nbuf = len(sems)
        wid = lax.axis_index("s") * _NC + lax.axis_index("c")
        base = wid * tok_per_w
        srcs = ((hist_hbm, x0), (post_hbm, x1), (auth_hbm, x2), (act_hbm, x3))
        for t, (idx_hbm, _) in enumerate(srcs):
            pltpu.sync_copy(idx_hbm.at[pl.ds(base, tok_per_w)], idx_v.at[t])
        jobs = [(t, c) for t in range(len(srcs)) for c in range(ngch)]

        def fire(i):
            t, c = jobs[i]
            pltpu.async_copy(
                embed_hbm.at[idx_v.at[t, pl.ds(c * _GCH, _GCH)]],
                rows_v.at[i % nbuf], sems[i % nbuf])

        def drain(i):
            t, c = jobs[i]
            pltpu.make_async_copy(
                embed_hbm.at[idx_v.at[t, pl.ds(c * _GCH, _GCH)]],
                rows_v.at[i % nbuf], sems[i % nbuf]).wait()
            pltpu.sync_copy(rows_v.at[i % nbuf],
                            srcs[t][1].at[pl.ds(base + c * _GCH, _GCH)])

        for i in range(nbuf - 1):
            fire(i)
        for i in range(len(jobs)):
            if i + nbuf - 1 < len(jobs):
                fire(i + nbuf - 1)
            drain(i)

    mesh = plsc.VectorSubcoreMesh(core_axis_name="c", subcore_axis_name="s")
    xt = jax.ShapeDtypeStruct((nt, H), jnp.float32)
    return functools.partial(
        pl.kernel, mesh=mesh,
        out_type=[xt, xt, xt, xt],
        compiler_params=pltpu.CompilerParams(needs_layout_passes=False),
        scratch_types=[
            pltpu.VMEM((4, tok_per_w), jnp.int32),
            pltpu.VMEM((4, _GCH, H), jnp.float32),
            pltpu.SemaphoreType.DMA,
            pltpu.SemaphoreType.DMA,
            pltpu.SemaphoreType.DMA,
            pltpu.SemaphoreType.DMA,
        ],
    )(body)


_BT = 2048


def _tc_mlp_body(x0, x1, x2, x3, gap, tgp, gamma, beta, w1, b1, w2, b2, out):
    oh = jnp.where(gap[...] == lax.broadcasted_iota(jnp.int32, (_BT, H), 1),
                   1.0, 0.0)
    x4 = jnp.dot(oh, tgp[...], preferred_element_type=jnp.float32,
                 precision=lax.Precision.HIGHEST)
    x = jnp.concatenate(
        [x0[...], x1[...], x2[...], x3[...], x4], axis=1)
    mu = jnp.mean(x, axis=-1, keepdims=True)
    var = jnp.mean((x - mu) ** 2, axis=-1, keepdims=True)
    xn = (x - mu) * lax.rsqrt(var + 1e-5) * gamma[...] + beta[...]
    h1 = jnp.dot(xn, w1[...], preferred_element_type=jnp.float32) + b1[...]
    h1 = h1 * jax.nn.sigmoid(h1)
    ev = jnp.dot(h1, w2[...], preferred_element_type=jnp.float32) + b2[...]
    out[...] = ev


def _tc_mlp(xs, gap2d, tgpad, ln_gamma, ln_beta, W1, b1, W2, b2, nt=NT):
    D5 = 5 * H
    D4 = 4 * H
    grid = (nt // _BT,)
    xspec = pl.BlockSpec((_BT, H), lambda i: (i, 0))

    def full(shape):
        return pl.BlockSpec(shape, lambda i: tuple(0 for _ in shape))

    return pl.pallas_call(
        _tc_mlp_body,
        grid=grid,
        in_specs=[xspec] * 4 + [pl.BlockSpec((_BT, 1), lambda i: (i, 0)),
                                full((H, H)), full((1, D5)), full((1, D5)),
                                full((D5, D4)), full((1, D4)),
                                full((D4, H)), full((1, H))],
        out_specs=pl.BlockSpec((_BT, H), lambda i: (i, 0)),
        out_shape=jax.ShapeDtypeStruct((nt, H), jnp.float32),
    )(*xs, gap2d, tgpad, ln_gamma.reshape(1, D5), ln_beta.reshape(1, D5),
      W1, b1.reshape(1, D4), W2, b2.reshape(1, H))


_PCH = 128
_NPCH = K // _PCH
_NVL = L // _LANES
_PAIRS_PER_W = NPAIR // _NW


def _sc_pack_body(gid_hbm, msk_hbm, event_hbm, emp_hbm,
                  states_hbm, mout_hbm,
                  ids_v, msk_v, idxl_v, chunk_v, emp_v, mko_v, sem):
    wid = lax.axis_index("s") * _NC + lax.axis_index("c")
    iota = lax.iota(jnp.int32, _LANES)
    zero16 = jnp.zeros((_LANES,), jnp.int32)

    for p in range(_PAIRS_PER_W):
        pair = wid + p * _NW
        b = pair // S
        s = pair % S
        pltpu.sync_copy(gid_hbm.at[b], ids_v)
        pltpu.sync_copy(msk_hbm.at[b], msk_v)
        pltpu.sync_copy(emp_hbm.at[s], emp_v)

        sval = s + 1

        def count_body(j, c):
            ids16 = ids_v[pl.ds(j * _LANES, _LANES)]
            mk16 = msk_v[pl.ds(j * _LANES, _LANES)]
            m = (ids16 == sval) & (mk16 != 0)
            return c + jnp.sum(jnp.where(m, 1, 0))

        count = lax.fori_loop(0, _NVL, count_body, jnp.int32(0), unroll=4)
        start = jnp.maximum(count - K, 0)
        n = count - start

        def zidx_body(j, _):
            idxl_v[pl.ds(j * _LANES, _LANES)] = j * _LANES + iota
            return 0

        lax.fori_loop(0, K // _LANES, zidx_body, 0)

        def rank_body(j, c):
            ids16 = ids_v[pl.ds(j * _LANES, _LANES)]
            mk16 = msk_v[pl.ds(j * _LANES, _LANES)]
            m = (ids16 == sval) & (mk16 != 0)
            mi = jnp.where(m, 1, 0)
            rank = plsc.cumsum(mi) + c - 1
            slot = rank - start
            wm = m & (slot >= 0)
            slot_c = jnp.maximum(slot, 0)
            gidx = b * L + j * _LANES + iota
            plsc.store_scatter(idxl_v, [slot_c], gidx, mask=wm)
            return c + jnp.sum(mi)

        lax.fori_loop(0, _NVL, rank_body, jnp.int32(0), unroll=2)

        for c4 in range(_NPCH):
            k0 = c4 * _PCH

            @pl.when(k0 < n)
            def _(c4=c4, k0=k0):
                pltpu.async_copy(event_hbm.at[idxl_v.at[pl.ds(k0, _PCH)]],
                                 chunk_v.at[c4], sem)

        for c4 in range(_NPCH):
            k0 = c4 * _PCH

            @pl.when(k0 < n)
            def _(c4=c4, k0=k0):
                pltpu.make_async_copy(
                    event_hbm.at[idxl_v.at[pl.ds(k0, _PCH)]],
                    chunk_v.at[c4], sem).wait()

            if c4 == 0:
                @pl.when(n == 0)
                def _():
                    for j2 in range(H // _LANES):
                        sl = pl.ds(j2 * _LANES, _LANES)
                        chunk_v[0, 0, sl] = emp_v[sl]

                pltpu.sync_copy(chunk_v.at[0],
                                states_hbm.at[pl.ds(pair * K, _PCH)])
            else:
                @pl.when(k0 < n)
                def _(c4=c4, k0=k0):
                    pltpu.sync_copy(chunk_v.at[c4],
                                    states_hbm.at[pl.ds(pair * K + k0, _PCH)])

        def mk_body(j, _):
            k16 = j * _LANES + iota
            mv = (k16 < n) | ((k16 == 0) & (n == 0))
            mko_v[pl.ds(j * _LANES, _LANES)] = jnp.where(mv, 1, 0)
            return 0

        lax.fori_loop(0, K // _LANES, mk_body, 0)
        pltpu.sync_copy(mko_v, mout_hbm.at[pair])


def _sc_pack(gid, maskI, event, empty_tokens):
    mesh = plsc.VectorSubcoreMesh(core_axis_name="c", subcore_axis_name="s")
    fn = functools.partial(
        pl.kernel, mesh=mesh,
        out_type=[jax.ShapeDtypeStruct((NPAIR * K, H), jnp.float32),
                  jax.ShapeDtypeStruct((NPAIR, K), jnp.int32)],
        compiler_params=pltpu.CompilerParams(needs_layout_passes=False),
        scratch_types=[
            pltpu.VMEM((L,), jnp.int32),
            pltpu.VMEM((L,), jnp.int32),
            pltpu.VMEM((K,), jnp.int32),
            pltpu.VMEM((_NPCH, _PCH, H), jnp.float32),
            pltpu.VMEM((H,), jnp.float32),
            pltpu.VMEM((K,), jnp.int32),
            pltpu.SemaphoreType.DMA,
        ],
    )(_sc_pack_body)
    return fn(gid, maskI, event, empty_tokens)


def _tc_final_body(raw, maskf, pos, sid, out):
    x = raw[0]
    m = maskf[0]
    val = x + pos[...][None, :, :] + sid[...][:, None, :]
    out[0] = jnp.where(m > 0.0, val, 0.0)


def _tc_final(raw, maskf, pos_table, sid_rows):
    return pl.pallas_call(
        _tc_final_body,
        grid=(B,),
        in_specs=[
            pl.BlockSpec((1, S, K, H), lambda i: (i, 0, 0, 0)),
            pl.BlockSpec((1, S, K, 1), lambda i: (i, 0, 0, 0)),
            pl.BlockSpec((K, H), lambda i: (0, 0)),
            pl.BlockSpec((S, H), lambda i: (0, 0)),
        ],
        out_specs=pl.BlockSpec((1, S, K, H), lambda i: (i, 0, 0, 0)),
        out_shape=jax.ShapeDtypeStruct((B, S, K, H), jnp.float32),
    )(raw, maskf, pos_table, sid_rows)


def kernel(history_tokens, history_post_tokens, history_author_tokens,
           history_action_tokens, history_time_gap, history_group_ids,
           history_mask, embed_table, time_gap_table, seq_id_table, pos_table,
           ln_gamma, ln_beta, W1, b1, W2, b2, empty_tokens):
    hist = history_tokens.reshape(NT).astype(jnp.int32)
    post = history_post_tokens.reshape(NT).astype(jnp.int32)
    auth = history_author_tokens.reshape(NT).astype(jnp.int32)
    act = history_action_tokens.reshape(NT).astype(jnp.int32)
    gap = history_time_gap.reshape(NT).astype(jnp.int32)
    gid = history_group_ids.astype(jnp.int32)
    maskI = history_mask.astype(jnp.int32)

    tgpad = jnp.zeros((H, H), jnp.float32).at[:TG + 1].set(time_gap_table)
    xs = _make_sc_gather(NT)(embed_table, hist, post, auth, act)
    event = _tc_mlp(xs, gap.reshape(NT, 1), tgpad,
                    ln_gamma, ln_beta, W1, b1, W2, b2)
    sid_rows = seq_id_table[1:S + 1]
    raw, mout = _sc_pack(gid, maskI, event, empty_tokens)
    maskf = mout.astype(jnp.float32).reshape(B, S, K, 1)
    states = _tc_final(raw.reshape(B, S, K, H), maskf, pos_table, sid_rows)
    seq_mask = (mout != 0).reshape(B, S, K)
    return states, seq_mask
